# baseline (speedup 1.0000x reference)
import functools

import jax
import jax.numpy as jnp
from jax import lax
from jax.experimental import pallas as pl
from jax.experimental.pallas import tpu as pltpu
from jax.experimental.pallas import tpu_sc as plsc


_ROWS = 16384
_COLS = 1024
_NW = 32
_ROWS_W = _ROWS // _NW
_CR = 16
_STEPS = _ROWS_W // _CR
_NBUF = 3
_GROUPS = _STEPS // _NBUF
_TAIL = _STEPS - _GROUPS * _NBUF
_LANES = 16
_UNROLL = 8


def _compute_chunk(src, dst):
    def body(i, c):
        base = i * (_LANES * _UNROLL)
        for u in range(_UNROLL):
            off = base + u * _LANES
            for r in range(_CR):
                v = src[r, pl.ds(off, _LANES)]
                dst[r, pl.ds(off, _LANES)] = v.astype(jnp.int32).astype(
                    jnp.float32)
        return c

    lax.fori_loop(0, _COLS // (_LANES * _UNROLL), body, 0)


@functools.partial(
    pl.kernel,
    mesh=plsc.VectorSubcoreMesh(core_axis_name="c", subcore_axis_name="s"),
    out_type=jax.ShapeDtypeStruct((_ROWS, _COLS), jnp.float32),
    scratch_types=(
        [pltpu.VMEM((_NBUF, _CR, _COLS), jnp.float32),
         pltpu.VMEM((_NBUF, _CR, _COLS), jnp.float32)]
        + [pltpu.SemaphoreType.DMA] * (2 * _NBUF)
    ),
)
def _sc_trunc(x_hbm, out_hbm, ibuf, obuf, *sems):
    isems = sems[:_NBUF]
    osems = sems[_NBUF:]
    wid = lax.axis_index("s") * 2 + lax.axis_index("c")
    base = wid * _ROWS_W

    def in_slice(j):
        return x_hbm.at[pl.ds(base + j * _CR, _CR), :]

    def out_slice(j):
        return out_hbm.at[pl.ds(base + j * _CR, _CR), :]

    def wait_in(j, b):
        pltpu.make_async_copy(in_slice(j), ibuf.at[b], isems[b]).wait()

    def wait_out(j, b):
        pltpu.make_async_copy(obuf.at[b], out_slice(j), osems[b]).wait()

    for b in range(_NBUF):
        pltpu.async_copy(in_slice(b), ibuf.at[b], isems[b])

    def group(g, c):
        for b in range(_NBUF):
            j = g * _NBUF + b
            wait_in(j, b)

            @pl.when(g > 0)
            def _():
                wait_out(j - _NBUF, b)

            _compute_chunk(ibuf.at[b], obuf.at[b])
            pltpu.async_copy(obuf.at[b], out_slice(j), osems[b])

            @pl.when(j + _NBUF < _STEPS)
            def _():
                pltpu.async_copy(in_slice(j + _NBUF), ibuf.at[b], isems[b])
        return c

    lax.fori_loop(0, _GROUPS, group, 0)

    for t in range(_TAIL):
        j = _GROUPS * _NBUF + t
        b = j % _NBUF
        wait_in(j, b)
        wait_out(j - _NBUF, b)
        _compute_chunk(ibuf.at[b], obuf.at[b])
        pltpu.async_copy(obuf.at[b], out_slice(j), osems[b])

    for j in range(_STEPS - _NBUF, _STEPS):
        wait_out(j, j % _NBUF)


def kernel(x):
    return _sc_trunc(x)


# speedup vs baseline: 1.7724x; 1.7724x over previous
import functools

import jax
import jax.numpy as jnp
from jax import lax
from jax.experimental import pallas as pl
from jax.experimental.pallas import tpu as pltpu
from jax.experimental.pallas import tpu_sc as plsc


_ROWS = 16384
_COLS = 1024
_NW = 32
_ROWS_W = _ROWS // _NW
_CR = 8
_STEPS = _ROWS_W // _CR
_NBUF = 6
_GROUPS = _STEPS // _NBUF
_TAIL = _STEPS - _GROUPS * _NBUF
_LANES = 16
_UNROLL = 8


def _compute_chunk(src, dst):
    def body(i, c):
        base = i * (_LANES * _UNROLL)
        for u in range(_UNROLL):
            off = base + u * _LANES
            for r in range(_CR):
                v = src[r, pl.ds(off, _LANES)]
                dst[r, pl.ds(off, _LANES)] = v.astype(jnp.int32).astype(
                    jnp.float32)
        return c

    lax.fori_loop(0, _COLS // (_LANES * _UNROLL), body, 0)


@functools.partial(
    pl.kernel,
    mesh=plsc.VectorSubcoreMesh(core_axis_name="c", subcore_axis_name="s"),
    out_type=jax.ShapeDtypeStruct((_ROWS, _COLS), jnp.float32),
    scratch_types=(
        [pltpu.VMEM((_NBUF, _CR, _COLS), jnp.float32),
         pltpu.VMEM((_NBUF, _CR, _COLS), jnp.float32)]
        + [pltpu.SemaphoreType.DMA] * (2 * _NBUF)
    ),
)
def _sc_trunc(x_hbm, out_hbm, ibuf, obuf, *sems):
    isems = sems[:_NBUF]
    osems = sems[_NBUF:]
    wid = lax.axis_index("s") * 2 + lax.axis_index("c")
    base = wid * _ROWS_W

    def in_slice(j):
        return x_hbm.at[pl.ds(base + j * _CR, _CR), :]

    def out_slice(j):
        return out_hbm.at[pl.ds(base + j * _CR, _CR), :]

    for b in range(_NBUF):
        pltpu.async_copy(in_slice(b), ibuf.at[b], isems[b])

    def group(g, c):
        for b in range(_NBUF):
            j = g * _NBUF + b
            pltpu.make_async_copy(in_slice(j), ibuf.at[b], isems[b]).wait()

            @pl.when(g > 0)
            def _():
                pltpu.make_async_copy(
                    obuf.at[b], out_slice(j - _NBUF), osems[b]).wait()

            _compute_chunk(ibuf.at[b], obuf.at[b])
            pltpu.async_copy(obuf.at[b], out_slice(j), osems[b])

            @pl.when(j + _NBUF < _STEPS)
            def _():
                pltpu.async_copy(in_slice(j + _NBUF), ibuf.at[b], isems[b])
        return c

    lax.fori_loop(0, _GROUPS, group, 0)

    for t in range(_TAIL):
        j = _GROUPS * _NBUF + t
        b = j % _NBUF
        pltpu.make_async_copy(in_slice(j), ibuf.at[b], isems[b]).wait()
        pltpu.make_async_copy(
            obuf.at[b], out_slice(j - _NBUF), osems[b]).wait()
        _compute_chunk(ibuf.at[b], obuf.at[b])
        pltpu.async_copy(obuf.at[b], out_slice(j), osems[b])

    for j in range(_STEPS - _NBUF, _STEPS):
        pltpu.make_async_copy(
            obuf.at[j % _NBUF], out_slice(j), osems[j % _NBUF]).wait()


def kernel(x):
    return _sc_trunc(x)


# speedup vs baseline: 1.9416x; 1.0955x over previous
import functools

import jax
import jax.numpy as jnp
from jax import lax
from jax.experimental import pallas as pl
from jax.experimental.pallas import tpu as pltpu
from jax.experimental.pallas import tpu_sc as plsc


_ROWS = 16384
_COLS = 1024
_NW = 32
_ROWS_W = _ROWS // _NW
_CR = 4
_STEPS = _ROWS_W // _CR
_NBUF = 8
_GROUPS = _STEPS // _NBUF
_LANES = 16
_UNROLL = 8


def _compute_chunk(src, dst):
    def body(i, c):
        base = i * (_LANES * _UNROLL)
        for u in range(_UNROLL):
            off = base + u * _LANES
            for r in range(_CR):
                v = src[r, pl.ds(off, _LANES)]
                dst[r, pl.ds(off, _LANES)] = v.astype(jnp.int32).astype(
                    jnp.float32)
        return c

    lax.fori_loop(0, _COLS // (_LANES * _UNROLL), body, 0)


@functools.partial(
    pl.kernel,
    mesh=plsc.VectorSubcoreMesh(core_axis_name="c", subcore_axis_name="s"),
    out_type=jax.ShapeDtypeStruct((_ROWS, _COLS), jnp.float32),
    scratch_types=(
        [pltpu.VMEM((_NBUF, _CR, _COLS), jnp.float32),
         pltpu.VMEM((_NBUF, _CR, _COLS), jnp.float32)]
        + [pltpu.SemaphoreType.DMA] * (2 * _NBUF)
    ),
)
def _sc_trunc(x_hbm, out_hbm, ibuf, obuf, *sems):
    isems = sems[:_NBUF]
    osems = sems[_NBUF:]
    wid = lax.axis_index("s") * 2 + lax.axis_index("c")
    base = wid * _ROWS_W

    def in_slice(j):
        return x_hbm.at[pl.ds(base + j * _CR, _CR), :]

    def out_slice(j):
        return out_hbm.at[pl.ds(base + j * _CR, _CR), :]

    for b in range(_NBUF):
        pltpu.async_copy(in_slice(b), ibuf.at[b], isems[b])

    def group(g, c):
        for b in range(_NBUF):
            j = g * _NBUF + b
            pltpu.make_async_copy(in_slice(j), ibuf.at[b], isems[b]).wait()

            @pl.when(g > 0)
            def _():
                pltpu.make_async_copy(
                    obuf.at[b], out_slice(j - _NBUF), osems[b]).wait()

            _compute_chunk(ibuf.at[b], obuf.at[b])
            pltpu.async_copy(obuf.at[b], out_slice(j), osems[b])

            @pl.when(j + _NBUF < _STEPS)
            def _():
                pltpu.async_copy(in_slice(j + _NBUF), ibuf.at[b], isems[b])
        return c

    lax.fori_loop(0, _GROUPS, group, 0)

    for b in range(_NBUF):
        pltpu.make_async_copy(
            obuf.at[b], out_slice(_STEPS - _NBUF + b), osems[b]).wait()


def kernel(x):
    return _sc_trunc(x)
